# baseline (speedup 1.0000x reference)
import numpy as np

import jax
import jax.numpy as jnp
from jax.experimental import pallas as pl
from jax.experimental.pallas import tpu as pltpu
from jax.experimental.pallas import tpu_sc as plsc

_F = 2000
_E = 3 * _F
_EPAD = 6144
_B = 1024
_NB = _EPAD // _B
_DIST2 = (1.0 + 1e-6) ** 2
_CROSS2 = (1e-5 - 1e-8) ** 2
_FPAD = 2048

_TRI = [(i, j) for i in range(_NB) for j in range(i, _NB)]
_NT = len(_TRI)
_TRI_I = np.array([ij[0] for ij in _TRI], dtype=np.int32)
_TRI_J = np.array([ij[1] for ij in _TRI], dtype=np.int32)


_NW = 32
_BPW = (2 * _EPAD) // _NW


def _sc_gather_kernel(table_hbm, idx_hbm, out_hbm, idx_v, rows_v, sem):
    wid = jax.lax.axis_index("s") * 2 + jax.lax.axis_index("c")
    base = wid * _BPW
    pltpu.sync_copy(idx_hbm.at[pl.ds(base, _BPW)], idx_v)
    pltpu.async_copy(table_hbm.at[idx_v], rows_v, sem).wait()
    pltpu.sync_copy(rows_v, out_hbm.at[pl.ds(base, _BPW)])


def _pair_kernel(im_ref, jm_ref, rdat_ref, cdat_ref, row_ref, col_ref):
    t = pl.program_id(0)
    iv = im_ref[t]
    jv = jm_ref[t]
    r = rdat_ref[...]
    c = cdat_ref[...]
    sp = jnp.dot(r[:, 0:5], c[0:5, :], preferred_element_type=jnp.float32)
    gd = jnp.dot(r[:, 5:8], c[5:8, :], preferred_element_type=jnp.float32)
    nbr = r[:, 8:9].astype(jnp.float32)
    nbc = c[8:9, :].astype(jnp.float32)
    q = (gd * gd + _CROSS2) - nbr * nbc
    m = (jnp.maximum(sp, q) < 0.0).astype(jnp.float32)
    rowpart = jnp.sum(m, axis=1, keepdims=True)

    @pl.when(t == 0)
    def _zero_cols():
        col_ref[...] = jnp.zeros_like(col_ref)

    @pl.when(iv == jv)
    def _diag():
        row_ref[...] = rowpart

    @pl.when(jv > iv)
    def _upper():
        row_ref[...] += rowpart
        col_ref[:, pl.ds(jv * _B, _B)] += jnp.sum(m, axis=0, keepdims=True)


def _loss_kernel(n0_ref, n1_ref, n2_ref, fp_ref, out_ref):
    cc = n0_ref[...] + n1_ref[...] + n2_ref[...]
    cc = jnp.clip(cc, 0.0, 100.0)
    out_ref[...] = (jnp.sum(cc * fp_ref[...]) / _F)[None, None]


def kernel(vertices, faces, face_probs):
    f0 = faces[:, 0]
    f1 = faces[:, 1]
    f2 = faces[:, 2]
    starts = jnp.concatenate([f0, f1, f2])
    ends = jnp.concatenate([f1, f2, f0])

    vpad = jnp.pad(vertices, ((0, 0), (0, 125)))
    idx_all = jnp.concatenate([
        jnp.pad(starts, (0, _EPAD - _E)),
        jnp.pad(ends, (0, _EPAD - _E))])
    mesh = plsc.VectorSubcoreMesh(core_axis_name="c", subcore_axis_name="s")
    gathered = pl.kernel(
        _sc_gather_kernel,
        mesh=mesh,
        out_type=jax.ShapeDtypeStruct((2 * _EPAD, 128), jnp.float32),
        scratch_types=[
            pltpu.VMEM((_BPW,), jnp.int32),
            pltpu.VMEM((_BPW, 128), jnp.float32),
            pltpu.SemaphoreType.DMA,
        ],
    )(vpad, idx_all)
    p0 = gathered[:_E, 0:3]
    p1 = gathered[_EPAD:_EPAD + _E, 0:3]
    cen = (p0 + p1) * 0.5
    d = (p1 - p0) + 1e-8
    nc = jnp.sum(cen * cen, axis=1, keepdims=True)
    nd = jnp.sum(d * d, axis=1, keepdims=True)
    one = jnp.ones_like(nc)
    rdat = jnp.concatenate([cen, nc, one, d, nd, one], axis=1)
    rdat = jnp.pad(rdat, ((0, _EPAD - _E), (0, 0)))
    cdat = jnp.concatenate(
        [-2.0 * cen, one, nc - _DIST2, d, nd, -_CROSS2 * one], axis=1)
    cdat = jnp.pad(cdat, ((0, _EPAD - _E), (0, 0)))
    cdat = cdat.at[_E:, 4].set(1e12)
    cdat = cdat.T
    rdat = rdat.astype(jnp.bfloat16)
    cdat = cdat.astype(jnp.bfloat16)

    nrow, ncol = pl.pallas_call(
        _pair_kernel,
        grid_spec=pltpu.PrefetchScalarGridSpec(
            num_scalar_prefetch=2,
            grid=(_NT,),
            in_specs=[
                pl.BlockSpec((_B, 10), lambda t, im, jm: (im[t], 0)),
                pl.BlockSpec((10, _B), lambda t, im, jm: (0, jm[t])),
            ],
            out_specs=[
                pl.BlockSpec((_B, 1), lambda t, im, jm: (im[t], 0)),
                pl.BlockSpec((1, _EPAD), lambda t, im, jm: (0, 0)),
            ],
        ),
        out_shape=[
            jax.ShapeDtypeStruct((_EPAD, 1), jnp.float32),
            jax.ShapeDtypeStruct((1, _EPAD), jnp.float32),
        ],
    )(jnp.asarray(_TRI_I), jnp.asarray(_TRI_J), rdat, cdat)

    n = nrow[:_E, 0] + ncol[0, :_E]
    n0 = jnp.pad(n[0::3], (0, _FPAD - _F))[None, :]
    n1 = jnp.pad(n[1::3], (0, _FPAD - _F))[None, :]
    n2 = jnp.pad(n[2::3], (0, _FPAD - _F))[None, :]
    fp = jnp.pad(face_probs, (0, _FPAD - _F))[None, :]

    loss = pl.pallas_call(
        _loss_kernel,
        out_shape=jax.ShapeDtypeStruct((1, 1), jnp.float32),
    )(n0, n1, n2, fp)
    return loss[0, 0]


# speedup vs baseline: 1.2791x; 1.2791x over previous
import numpy as np

import jax
import jax.numpy as jnp
from jax.experimental import pallas as pl
from jax.experimental.pallas import tpu as pltpu

_F = 2000
_E = 3 * _F
_EPAD = 6144
_B = 1024
_NB = _EPAD // _B
_DIST2 = (1.0 + 1e-6) ** 2
_CROSS2 = (1e-5 - 1e-8) ** 2
_FPAD = 2048

_TRI = [(i, j) for i in range(_NB) for j in range(i, _NB)]
_NT = len(_TRI)
_TRI_I = np.array([ij[0] for ij in _TRI], dtype=np.int32)
_TRI_J = np.array([ij[1] for ij in _TRI], dtype=np.int32)


def _pair_kernel(im_ref, jm_ref, rdat_ref, cdat_ref, row_ref, col_ref):
    t = pl.program_id(0)
    iv = im_ref[t]
    jv = jm_ref[t]
    r = rdat_ref[...]
    c = cdat_ref[...]
    sp = jnp.dot(r[:, 0:5], c[0:5, :], preferred_element_type=jnp.float32)
    gd = jnp.dot(r[:, 5:8], c[5:8, :], preferred_element_type=jnp.float32)
    nbr = r[:, 8:9].astype(jnp.float32)
    nbc = c[8:9, :].astype(jnp.float32)
    q = (gd * gd + _CROSS2) - nbr * nbc
    m = (jnp.maximum(sp, q) < 0.0).astype(jnp.float32)
    rowpart = jnp.sum(m, axis=1, keepdims=True)

    @pl.when(t == 0)
    def _zero_cols():
        col_ref[...] = jnp.zeros_like(col_ref)

    @pl.when(iv == jv)
    def _diag():
        row_ref[...] = rowpart

    @pl.when(jv > iv)
    def _upper():
        row_ref[...] += rowpart
        col_ref[:, pl.ds(jv * _B, _B)] += jnp.sum(m, axis=0, keepdims=True)


def _loss_kernel(n0_ref, n1_ref, n2_ref, fp_ref, out_ref):
    cc = n0_ref[...] + n1_ref[...] + n2_ref[...]
    cc = jnp.clip(cc, 0.0, 100.0)
    out_ref[...] = (jnp.sum(cc * fp_ref[...]) / _F)[None, None]


def kernel(vertices, faces, face_probs):
    f0 = faces[:, 0]
    f1 = faces[:, 1]
    f2 = faces[:, 2]
    starts = jnp.concatenate([f0, f1, f2])
    ends = jnp.concatenate([f1, f2, f0])
    p0 = vertices[starts]
    p1 = vertices[ends]
    cen = (p0 + p1) * 0.5
    d = (p1 - p0) + 1e-8
    nc = jnp.sum(cen * cen, axis=1, keepdims=True)
    nd = jnp.sum(d * d, axis=1, keepdims=True)
    one = jnp.ones_like(nc)
    rdat = jnp.concatenate([cen, nc, one, d, nd, one], axis=1)
    rdat = jnp.pad(rdat, ((0, _EPAD - _E), (0, 0)))
    cdat = jnp.concatenate(
        [-2.0 * cen, one, nc - _DIST2, d, nd, -_CROSS2 * one], axis=1)
    cdat = jnp.pad(cdat, ((0, _EPAD - _E), (0, 0)))
    cdat = cdat.at[_E:, 4].set(1e12)
    cdat = cdat.T
    rdat = rdat.astype(jnp.bfloat16)
    cdat = cdat.astype(jnp.bfloat16)

    nrow, ncol = pl.pallas_call(
        _pair_kernel,
        grid_spec=pltpu.PrefetchScalarGridSpec(
            num_scalar_prefetch=2,
            grid=(_NT,),
            in_specs=[
                pl.BlockSpec((_B, 10), lambda t, im, jm: (im[t], 0)),
                pl.BlockSpec((10, _B), lambda t, im, jm: (0, jm[t])),
            ],
            out_specs=[
                pl.BlockSpec((_B, 1), lambda t, im, jm: (im[t], 0)),
                pl.BlockSpec((1, _EPAD), lambda t, im, jm: (0, 0)),
            ],
        ),
        out_shape=[
            jax.ShapeDtypeStruct((_EPAD, 1), jnp.float32),
            jax.ShapeDtypeStruct((1, _EPAD), jnp.float32),
        ],
    )(jnp.asarray(_TRI_I), jnp.asarray(_TRI_J), rdat, cdat)

    n = nrow[:_E, 0] + ncol[0, :_E]
    n0 = jnp.pad(n[0::3], (0, _FPAD - _F))[None, :]
    n1 = jnp.pad(n[1::3], (0, _FPAD - _F))[None, :]
    n2 = jnp.pad(n[2::3], (0, _FPAD - _F))[None, :]
    fp = jnp.pad(face_probs, (0, _FPAD - _F))[None, :]

    loss = pl.pallas_call(
        _loss_kernel,
        out_shape=jax.ShapeDtypeStruct((1, 1), jnp.float32),
    )(n0, n1, n2, fp)
    return loss[0, 0]
